# Initial kernel scaffold; baseline (speedup 1.0000x reference)
import jax
import jax.numpy as jnp
from jax.experimental import pallas as pl


def kernel(hidden_states, codebook_hidden_states, Wq, bq, Wk, bk, Wv, bv, Wp, bp):
    raise NotImplementedError("write your pallas kernel here")


# speedup vs baseline: 1.6056x; 1.6056x over previous
import functools
import math

import jax
import jax.numpy as jnp
from jax import lax
from jax.experimental import pallas as pl

B, C, T, N, H = 16, 512, 576, 1024, 4
DH = C // H
SF = 1.0 / math.sqrt(DH)
INV_SQRT_H = 1.0 / math.sqrt(H)


def _bf16_round(x):
    return lax.convert_element_type(
        lax.convert_element_type(x, jnp.bfloat16), jnp.float32)


def _prep_body(cb_ref, wk_ref, bk_ref, wv_ref, bv_ref, kf_ref, val_ref):
    cb = cb_ref[...]
    kf_ref[...] = lax.dot_general(
        cb, wk_ref[...], (((1,), (1,)), ((), ())),
        preferred_element_type=jnp.float32) + bk_ref[...]
    val = lax.dot_general(
        cb, wv_ref[...], (((1,), (1,)), ((), ())),
        preferred_element_type=jnp.float32) + bv_ref[...]
    val_ref[...] = _bf16_round(val)


def _main_body(x_ref, wq_ref, bq_ref, wp_ref, bp_ref, kf_ref, val_ref,
               logits_ref, idx_ref, zq_ref):
    x = x_ref[0]
    q = lax.dot_general(x, wq_ref[...], (((0,), (1,)), ((), ())),
                        preferred_element_type=jnp.float32)
    q = q + bq_ref[...]
    cp = lax.dot_general(x, wp_ref[...], (((0,), (1,)), ((), ())),
                         preferred_element_type=jnp.float32)
    cp = jnp.transpose(cp + bp_ref[...])
    cpb = _bf16_round(cp)
    kf = kf_ref[...]
    acc = None
    for h in range(H):
        q_h = q[:, h * DH:(h + 1) * DH]
        k_h = kf[:, h * DH:(h + 1) * DH]
        l1 = lax.dot_general(k_h, q_h, (((1,), (1,)), ((), ())),
                             preferred_element_type=jnp.float32) * SF
        term = cpb[h:h + 1, :] * _bf16_round(l1)
        acc = term if acc is None else acc + term
    logits = acc * INV_SQRT_H
    logits_ref[0] = logits
    maxv = jnp.max(logits, axis=0, keepdims=True)
    iota = lax.broadcasted_iota(jnp.int32, (N, T), 0)
    cand = jnp.where(logits == maxv, iota, N)
    idx = jnp.min(cand, axis=0, keepdims=True)
    idx_ref[0] = idx
    onehot = (iota == idx).astype(jnp.float32)
    zq_ref[0] = lax.dot_general(
        val_ref[...], onehot, (((0,), (0,)), ((), ())),
        preferred_element_type=jnp.float32)


def kernel(hidden_states, codebook_hidden_states, Wq, bq, Wk, bk, Wv, bv,
           Wp, bp):
    bk2 = bk.reshape(1, C)
    bv2 = bv.reshape(1, C)
    bq2 = bq.reshape(1, C)
    bp2 = bp.reshape(1, H)

    keyflat, value = pl.pallas_call(
        _prep_body,
        out_shape=(
            jax.ShapeDtypeStruct((N, C), jnp.float32),
            jax.ShapeDtypeStruct((N, C), jnp.float32),
        ),
    )(codebook_hidden_states, Wk, bk2, Wv, bv2)

    full = lambda shape: pl.BlockSpec(shape, lambda b: (0,) * len(shape))
    logits, idx, zq = pl.pallas_call(
        _main_body,
        grid=(B,),
        in_specs=[
            pl.BlockSpec((1, C, T), lambda b: (b, 0, 0)),
            full((C, C)),
            full((1, C)),
            full((H, C)),
            full((1, H)),
            full((N, C)),
            full((N, C)),
        ],
        out_specs=(
            pl.BlockSpec((1, N, T), lambda b: (b, 0, 0)),
            pl.BlockSpec((1, 1, T), lambda b: (b, 0, 0)),
            pl.BlockSpec((1, C, T), lambda b: (b, 0, 0)),
        ),
        out_shape=(
            jax.ShapeDtypeStruct((B, N, T), jnp.float32),
            jax.ShapeDtypeStruct((B, 1, T), jnp.int32),
            jax.ShapeDtypeStruct((B, C, T), jnp.float32),
        ),
    )(hidden_states, Wq, bq2, Wp, bp2, keyflat, value)

    return (logits, idx, zq)
